# baseline (speedup 1.0000x reference)
import jax
import jax.numpy as jnp
from jax.experimental import pallas as pl

_NQ = 8
_K = 1024
_D = 64
_BLK = 64


def _pq_body(x_ref, cb_ref, q_ref, ids_ref):
    ids_cols = []
    for q in range(_NQ):
        xq = x_ref[:, q * _D:(q + 1) * _D]
        cb = cb_ref[q]
        cnorm = jnp.sum(cb * cb, axis=1)
        scores = cnorm[None, :] - 2.0 * jnp.dot(
            xq, cb.T, preferred_element_type=jnp.float32)
        ids = jnp.argmin(scores, axis=1).astype(jnp.int32)
        onehot = (jax.lax.broadcasted_iota(jnp.int32, scores.shape, 1)
                  == ids[:, None]).astype(jnp.float32)
        qv = jnp.dot(onehot, cb, preferred_element_type=jnp.float32)
        q_ref[:, q * _D:(q + 1) * _D] = qv
        ids_cols.append(ids[:, None])
    ids_ref[...] = jnp.concatenate(ids_cols, axis=1)


def kernel(x, codebooks):
    B = x.shape[0]
    grid = (B // _BLK,)
    q_out, ids_t = pl.pallas_call(
        _pq_body,
        grid=grid,
        in_specs=[
            pl.BlockSpec((_BLK, _NQ * _D), lambda i: (i, 0)),
            pl.BlockSpec((_NQ, _K, _D), lambda i: (0, 0, 0)),
        ],
        out_specs=[
            pl.BlockSpec((_BLK, _NQ * _D), lambda i: (i, 0)),
            pl.BlockSpec((_BLK, _NQ), lambda i: (i, 0)),
        ],
        out_shape=[
            jax.ShapeDtypeStruct((B, _NQ * _D), jnp.float32),
            jax.ShapeDtypeStruct((B, _NQ), jnp.int32),
        ],
    )(x, codebooks)
    return q_out, ids_t.T.astype(jnp.int64)


# speedup vs baseline: 78.1415x; 78.1415x over previous
import functools

import jax
import jax.numpy as jnp
from jax import lax
from jax.experimental import pallas as pl
from jax.experimental.pallas import tpu as pltpu
from jax.experimental.pallas import tpu_sc as plsc

_NQ = 8
_K = 1024
_D = 64
_BLK = 256

_NC = 2
_NS = 16
_NW = _NC * _NS
_ROWS = 8192 * _NQ
_RPW = _ROWS // _NW
_CH = 512
_IPD = 128
_DP = 128


def _score_body(x_ref, cbt_ref, ids_ref, gidx_ref):
    ids_cols = []
    gidx_cols = []
    for q in range(_NQ):
        xq = x_ref[:, q * _D:(q + 1) * _D]
        cbt = cbt_ref[q]
        cnorm = jnp.sum(cbt * cbt, axis=0, keepdims=True)
        scores = cnorm - 2.0 * jnp.dot(
            xq, cbt, preferred_element_type=jnp.float32)
        ids = jnp.argmin(scores, axis=1).astype(jnp.int32)
        ids_cols.append(ids[:, None])
        gidx_cols.append(ids[:, None] + q * _K)
    ids_ref[...] = jnp.concatenate(ids_cols, axis=1)
    gidx_ref[...] = jnp.concatenate(gidx_cols, axis=1)


def _sc_gather_body(table_ref, gidx_ref, out_ref, idx_v, rows_v, sem):
    wid = lax.axis_index("s") * _NC + lax.axis_index("c")
    base = wid * _RPW
    pltpu.sync_copy(gidx_ref.at[pl.ds(base, _RPW)], idx_v)
    for c in range(_RPW // _CH):
        cps = []
        for j in range(_CH // _IPD):
            off = c * _CH + j * _IPD
            cps.append(pltpu.async_copy(
                table_ref.at[idx_v.at[pl.ds(off, _IPD)]],
                rows_v.at[pl.ds(j * _IPD, _IPD)], sem))
        for cp in cps:
            cp.wait()
        pltpu.sync_copy(rows_v, out_ref.at[pl.ds(base + c * _CH, _CH)])


def kernel(x, codebooks):
    B = x.shape[0]
    cbt = codebooks.transpose(0, 2, 1)

    ids_bq, gidx_bq = pl.pallas_call(
        _score_body,
        grid=(B // _BLK,),
        in_specs=[
            pl.BlockSpec((_BLK, _NQ * _D), lambda i: (i, 0)),
            pl.BlockSpec((_NQ, _D, _K), lambda i: (0, 0, 0)),
        ],
        out_specs=[
            pl.BlockSpec((_BLK, _NQ), lambda i: (i, 0)),
            pl.BlockSpec((_BLK, _NQ), lambda i: (i, 0)),
        ],
        out_shape=[
            jax.ShapeDtypeStruct((B, _NQ), jnp.int32),
            jax.ShapeDtypeStruct((B, _NQ), jnp.int32),
        ],
    )(x, cbt)

    table = jnp.pad(codebooks.reshape(_NQ * _K, _D),
                    ((0, 0), (0, _DP - _D)))
    gidx_flat = gidx_bq.reshape(B * _NQ)

    sc_gather = functools.partial(
        pl.kernel,
        mesh=plsc.VectorSubcoreMesh(core_axis_name="c", subcore_axis_name="s"),
        out_type=jax.ShapeDtypeStruct((_ROWS, _DP), jnp.float32),
        scratch_types=[
            pltpu.VMEM((_RPW,), jnp.int32),
            pltpu.VMEM((_CH, _DP), jnp.float32),
            pltpu.SemaphoreType.DMA,
        ],
    )(_sc_gather_body)

    q_rows = sc_gather(table, gidx_flat)
    return (q_rows[:, :_D].reshape(B, _NQ * _D),
            ids_bq.T.astype(jnp.int64))


# speedup vs baseline: 130.4629x; 1.6696x over previous
import functools

import jax
import jax.numpy as jnp
from jax import lax
from jax.experimental import pallas as pl
from jax.experimental.pallas import tpu as pltpu
from jax.experimental.pallas import tpu_sc as plsc

_NQ = 8
_K = 1024
_D = 64
_BLK = 512

_NC = 2
_NS = 16
_NW = _NC * _NS
_ROWS = 8192 * _NQ
_RPW = _ROWS // _NW
_CH = 512
_IPD = 128
_DP = 128


def _score_body(x_ref, cbt_ref, ids_ref, gidx_ref):
    ids_cols = []
    gidx_cols = []
    for q in range(_NQ):
        xq = x_ref[:, q * _D:(q + 1) * _D]
        cbt = cbt_ref[q]
        cnorm = jnp.sum(cbt * cbt, axis=0, keepdims=True)
        scores = cnorm - jnp.dot(
            xq + xq, cbt, preferred_element_type=jnp.float32)
        ids = jnp.argmin(scores, axis=1).astype(jnp.int32)
        ids_cols.append(ids[:, None])
        gidx_cols.append(ids[:, None] + q * _K)
    ids_ref[...] = jnp.concatenate(ids_cols, axis=1)
    gidx_ref[...] = jnp.concatenate(gidx_cols, axis=1)


def _sc_gather_body(table_ref, gidx_ref, out_ref, idx_v, rows_v, sem):
    wid = lax.axis_index("s") * _NC + lax.axis_index("c")
    base = wid * _RPW
    pltpu.sync_copy(gidx_ref.at[pl.ds(base, _RPW)], idx_v)
    for c in range(_RPW // _CH):
        cps = []
        for j in range(_CH // _IPD):
            off = c * _CH + j * _IPD
            cps.append(pltpu.async_copy(
                table_ref.at[idx_v.at[pl.ds(off, _IPD)]],
                rows_v.at[pl.ds(j * _IPD, _IPD)], sem))
        for cp in cps:
            cp.wait()
        pltpu.sync_copy(rows_v, out_ref.at[pl.ds(base + c * _CH, _CH)])


def kernel(x, codebooks):
    B = x.shape[0]
    cbt = codebooks.transpose(0, 2, 1)

    ids_bq, gidx_bq = pl.pallas_call(
        _score_body,
        grid=(B // _BLK,),
        in_specs=[
            pl.BlockSpec((_BLK, _NQ * _D), lambda i: (i, 0)),
            pl.BlockSpec((_NQ, _D, _K), lambda i: (0, 0, 0)),
        ],
        out_specs=[
            pl.BlockSpec((_BLK, _NQ), lambda i: (i, 0)),
            pl.BlockSpec((_BLK, _NQ), lambda i: (i, 0)),
        ],
        out_shape=[
            jax.ShapeDtypeStruct((B, _NQ), jnp.int32),
            jax.ShapeDtypeStruct((B, _NQ), jnp.int32),
        ],
    )(x, cbt)

    table = jnp.pad(codebooks.reshape(_NQ * _K, _D),
                    ((0, 0), (0, _DP - _D)))
    gidx_flat = gidx_bq.reshape(B * _NQ)

    sc_gather = functools.partial(
        pl.kernel,
        mesh=plsc.VectorSubcoreMesh(core_axis_name="c", subcore_axis_name="s"),
        out_type=jax.ShapeDtypeStruct((_ROWS, _DP), jnp.float32),
        scratch_types=[
            pltpu.VMEM((_RPW,), jnp.int32),
            pltpu.VMEM((_CH, _DP), jnp.float32),
            pltpu.SemaphoreType.DMA,
        ],
    )(_sc_gather_body)

    q_rows = sc_gather(table, gidx_flat)
    return (q_rows[:, :_D].reshape(B, _NQ * _D),
            ids_bq.T.astype(jnp.int64))
